# Initial kernel scaffold; baseline (speedup 1.0000x reference)
import jax
import jax.numpy as jnp
from jax.experimental import pallas as pl


def kernel(mat, proj_dir, perm):
    raise NotImplementedError("write your pallas kernel here")


# speedup vs baseline: 8.3812x; 8.3812x over previous
import math

import jax
import jax.numpy as jnp
from jax.experimental import pallas as pl
from jax.experimental.pallas import tpu as pltpu

_NUM_PROJS = 8


def _lsh_body(x_ref, pd_ref, pt_ref, out_ref):
    x = x_ref[...]
    pd = pd_ref[...]
    pt = pt_ref[...]
    t = x.shape[0]

    y = jnp.dot(x, pd, preferred_element_type=jnp.float32)
    bits = (y > 0.0)

    lane4 = jax.lax.broadcasted_iota(jnp.int32, (1, 4), 1)
    pow_lo = jax.lax.shift_left(jnp.int32(1), lane4)
    lo = jnp.sum(jnp.where(bits[:, 0:4], pow_lo, 0), axis=1, keepdims=True)
    hi = jnp.sum(jnp.where(bits[:, 4:8], pow_lo, 0), axis=1, keepdims=True)

    lane16 = jax.lax.broadcasted_iota(jnp.int32, (t, 16), 1)
    oh_hi = (hi == lane16).astype(jnp.float32)
    rows = jnp.dot(oh_hi, pt, preferred_element_type=jnp.float32)
    vals = jnp.sum(jnp.where(lo == lane16, rows, 0.0), axis=1)

    ids = vals.astype(jnp.int32)
    out_ref[...] = ids.reshape(out_ref.shape)


def kernel(mat, proj_dir, perm):
    b, h, n, d = mat.shape
    m = b * h * n
    x = mat.reshape(m, d)
    pd = proj_dir.reshape(d, _NUM_PROJS)
    pt = perm.reshape(16, 16).astype(jnp.float32)

    tile = math.gcd(m, 4096)
    grid = m // tile

    out = pl.pallas_call(
        _lsh_body,
        grid=(grid,),
        in_specs=[
            pl.BlockSpec((tile, d), lambda i: (i, 0)),
            pl.BlockSpec((d, _NUM_PROJS), lambda i: (0, 0)),
            pl.BlockSpec((16, 16), lambda i: (0, 0)),
        ],
        out_specs=pl.BlockSpec((tile // 128, 128), lambda i: (i, 0)),
        out_shape=jax.ShapeDtypeStruct((m // 128, 128), jnp.int32),
        compiler_params=pltpu.CompilerParams(
            dimension_semantics=("arbitrary",),
        ),
    )(x, pd, pt)
    return out.reshape(b, h, n)


# speedup vs baseline: 8.7225x; 1.0407x over previous
import math

import jax
import jax.numpy as jnp
from jax.experimental import pallas as pl
from jax.experimental.pallas import tpu as pltpu

_NUM_PROJS = 8
_GRP = 16


def _lsh_body(x_ref, pd_ref, w2_ref, ptb_ref, g_ref, out_ref):
    x = x_ref[...]
    y = jnp.dot(x, pd_ref[...], preferred_element_type=jnp.float32)
    pm = jnp.where(y > 0.0, 1.0, -1.0)
    a = jnp.dot(pm, w2_ref[...], preferred_element_type=jnp.float32)
    oh = jnp.where(a == 4.0, 1.0, 0.0)
    oh_lo = oh[:, :256]
    oh_hi = oh[:, 256:]
    rows = jnp.dot(oh_hi, ptb_ref[...], preferred_element_type=jnp.float32)
    vals = jnp.dot(oh_lo * rows, g_ref[...],
                   preferred_element_type=jnp.float32)
    out_ref[...] = vals.astype(jnp.int32)


def kernel(mat, proj_dir, perm):
    b, h, n, d = mat.shape
    m = b * h * n
    rows_total = m // _GRP
    x = mat.reshape(rows_total, _GRP * d)
    pd = proj_dir.reshape(d, _NUM_PROJS).astype(jnp.float32)

    eye = jnp.eye(_GRP, dtype=jnp.float32)
    pd_big = jnp.kron(eye, pd)

    nib = jnp.arange(16, dtype=jnp.int32)
    hb = (2 * ((nib[None, :] >> jnp.arange(4, dtype=jnp.int32)[:, None]) & 1)
          - 1).astype(jnp.float32)
    pad_lo = jnp.concatenate([hb, jnp.zeros((4, 16), jnp.float32)], axis=0)
    pad_hi = jnp.concatenate([jnp.zeros((4, 16), jnp.float32), hb], axis=0)
    w2 = jnp.concatenate([jnp.kron(eye, pad_lo), jnp.kron(eye, pad_hi)],
                         axis=1)

    pt = perm.reshape(16, 16).astype(jnp.float32)
    ptb = jnp.kron(eye, pt)
    g = jnp.kron(eye, jnp.ones((16, 1), jnp.float32))

    tile_rows = math.gcd(rows_total, 256)
    grid = rows_total // tile_rows

    out = pl.pallas_call(
        _lsh_body,
        grid=(grid,),
        in_specs=[
            pl.BlockSpec((tile_rows, _GRP * d), lambda i: (i, 0)),
            pl.BlockSpec((_GRP * d, 128), lambda i: (0, 0)),
            pl.BlockSpec((128, 512), lambda i: (0, 0)),
            pl.BlockSpec((256, 256), lambda i: (0, 0)),
            pl.BlockSpec((256, _GRP), lambda i: (0, 0)),
        ],
        out_specs=pl.BlockSpec((tile_rows, _GRP), lambda i: (i, 0)),
        out_shape=jax.ShapeDtypeStruct((rows_total, _GRP), jnp.int32),
        compiler_params=pltpu.CompilerParams(
            dimension_semantics=("arbitrary",),
        ),
    )(x, pd_big, w2, ptb, g)
    return out.reshape(b, h, n)


# speedup vs baseline: 14.9411x; 1.7129x over previous
import math

import jax
import jax.numpy as jnp
from jax.experimental import pallas as pl
from jax.experimental.pallas import tpu as pltpu

_NUM_PROJS = 8


def _lsh_body(x_ref, pdt_ref, w2t_ref, ptt_ref, one_ref, out_ref):
    x = x_ref[...]
    yt = jax.lax.dot_general(pdt_ref[...], x, (((1,), (1,)), ((), ())),
                             preferred_element_type=jnp.float32)
    pm = jnp.where(yt > 0.0, 1.0, -1.0)
    a = jnp.dot(w2t_ref[...], pm, preferred_element_type=jnp.float32)
    oh = jnp.where(a == 4.0, 1.0, 0.0)
    rows = jnp.dot(ptt_ref[...], oh[16:32, :],
                   preferred_element_type=jnp.float32)
    vals = jnp.dot(one_ref[...], oh[0:16, :] * rows,
                   preferred_element_type=jnp.float32)
    out_ref[...] = vals.astype(jnp.int32).reshape(out_ref.shape)


def kernel(mat, proj_dir, perm):
    b, h, n, d = mat.shape
    m = b * h * n
    x = mat.reshape(m, d)
    pdt = proj_dir.reshape(d, _NUM_PROJS).astype(jnp.float32).T

    nib = jnp.arange(16, dtype=jnp.int32)
    hb = (2 * ((nib[None, :] >> jnp.arange(4, dtype=jnp.int32)[:, None]) & 1)
          - 1).astype(jnp.float32)
    zeros4 = jnp.zeros((4, 16), jnp.float32)
    w_lo = jnp.concatenate([hb, zeros4], axis=0)
    w_hi = jnp.concatenate([zeros4, hb], axis=0)
    w2t = jnp.concatenate([w_lo.T, w_hi.T], axis=0)

    ptt = perm.reshape(16, 16).astype(jnp.float32).T
    one = jnp.ones((1, 16), jnp.float32)

    tile = math.gcd(m, 4096)
    grid = m // tile

    out = pl.pallas_call(
        _lsh_body,
        grid=(grid,),
        in_specs=[
            pl.BlockSpec((tile, d), lambda i: (i, 0)),
            pl.BlockSpec((_NUM_PROJS, d), lambda i: (0, 0)),
            pl.BlockSpec((32, _NUM_PROJS), lambda i: (0, 0)),
            pl.BlockSpec((16, 16), lambda i: (0, 0)),
            pl.BlockSpec((1, 16), lambda i: (0, 0)),
        ],
        out_specs=pl.BlockSpec((1, 1, tile), lambda i: (i, 0, 0)),
        out_shape=jax.ShapeDtypeStruct((grid, 1, tile), jnp.int32),
        compiler_params=pltpu.CompilerParams(
            dimension_semantics=("arbitrary",),
        ),
    )(x, pdt, w2t, ptt, one)
    return out.reshape(b, h, n)


# speedup vs baseline: 33.4876x; 2.2413x over previous
import math

import jax
import jax.numpy as jnp
from jax.experimental import pallas as pl
from jax.experimental.pallas import tpu as pltpu

_NUM_PROJS = 8


def _lsh_body(xt_ref, pdt_ref, w2t_ref, ptt_ref, one_ref, out_ref):
    xt = xt_ref[0]
    yt = jnp.dot(pdt_ref[...], xt, preferred_element_type=jnp.float32)
    pm = jnp.where(yt > 0.0, 1.0, -1.0)
    a = jnp.dot(w2t_ref[...], pm, preferred_element_type=jnp.float32)
    oh = jnp.where(a == 4.0, 1.0, 0.0)
    rows = jnp.dot(ptt_ref[...], oh[16:32, :],
                   preferred_element_type=jnp.float32)
    vals = jnp.dot(one_ref[...], oh[0:16, :] * rows,
                   preferred_element_type=jnp.float32)
    ids = vals.astype(jnp.int32).reshape(vals.shape[1])
    out_ref[...] = ids.reshape(out_ref.shape)


def kernel(mat, proj_dir, perm):
    b, h, n, d = mat.shape
    m = b * h * n
    bh = b * h
    xt = mat.transpose(0, 1, 3, 2).reshape(bh, d, n)
    pdt = proj_dir.reshape(d, _NUM_PROJS).astype(jnp.float32).T

    nib = jnp.arange(16, dtype=jnp.int32)
    hb = (2 * ((nib[None, :] >> jnp.arange(4, dtype=jnp.int32)[:, None]) & 1)
          - 1).astype(jnp.float32)
    zeros4 = jnp.zeros((4, 16), jnp.float32)
    w_lo = jnp.concatenate([hb, zeros4], axis=0)
    w_hi = jnp.concatenate([zeros4, hb], axis=0)
    w2t = jnp.concatenate([w_lo.T, w_hi.T], axis=0)

    ptt = perm.reshape(16, 16).astype(jnp.float32).T
    one = jnp.ones((1, 16), jnp.float32)

    tile = math.gcd(n, 4096)
    grid_n = n // tile

    out = pl.pallas_call(
        _lsh_body,
        grid=(bh, grid_n),
        in_specs=[
            pl.BlockSpec((1, d, tile), lambda i, j: (i, 0, j)),
            pl.BlockSpec((_NUM_PROJS, d), lambda i, j: (0, 0)),
            pl.BlockSpec((32, _NUM_PROJS), lambda i, j: (0, 0)),
            pl.BlockSpec((16, 16), lambda i, j: (0, 0)),
            pl.BlockSpec((1, 16), lambda i, j: (0, 0)),
        ],
        out_specs=pl.BlockSpec((tile // 128, 128),
                               lambda i, j: (i * grid_n + j, 0)),
        out_shape=jax.ShapeDtypeStruct((m // 128, 128), jnp.int32),
        compiler_params=pltpu.CompilerParams(
            dimension_semantics=("arbitrary", "arbitrary"),
        ),
    )(xt, pdt, w2t, ptt, one)
    return out.reshape(b, h, n)


# speedup vs baseline: 53.3571x; 1.5933x over previous
import math

import jax
import jax.numpy as jnp
from jax.experimental import pallas as pl
from jax.experimental.pallas import tpu as pltpu

_NUM_PROJS = 8


def _make_body(k, d, n):
    def _lsh_body(xt_ref, pdt_ref, w2t_ref, ptt_ref, one_ref, out_ref):
        pdt = pdt_ref[...]
        w2t = w2t_ref[...]
        ptt = ptt_ref[...]
        one = one_ref[...]
        for s in range(k):
            xt = xt_ref[pl.ds(s * d, d), :]
            yt = jnp.dot(pdt, xt, preferred_element_type=jnp.float32)
            pm = jnp.where(yt > 0.0, 1.0, -1.0)
            a = jnp.dot(w2t, pm, preferred_element_type=jnp.float32)
            oh = jnp.where(a == 4.0, 1.0, 0.0)
            rows = jnp.dot(ptt, oh[16:32, :],
                           preferred_element_type=jnp.float32)
            vals = jnp.dot(one, oh[0:16, :] * rows,
                           preferred_element_type=jnp.float32)
            ids = vals.astype(jnp.int32).reshape(n)
            out_ref[pl.ds(s * (n // 128), n // 128), :] = ids.reshape(
                n // 128, 128)
    return _lsh_body


def kernel(mat, proj_dir, perm):
    b, h, n, d = mat.shape
    m = b * h * n
    bh = b * h
    xt = mat.transpose(0, 1, 3, 2).reshape(bh * d, n)
    pdt = proj_dir.reshape(d, _NUM_PROJS).astype(jnp.float32).T

    nib = jnp.arange(16, dtype=jnp.int32)
    hb = (2 * ((nib[None, :] >> jnp.arange(4, dtype=jnp.int32)[:, None]) & 1)
          - 1).astype(jnp.float32)
    zeros4 = jnp.zeros((4, 16), jnp.float32)
    w_lo = jnp.concatenate([hb, zeros4], axis=0)
    w_hi = jnp.concatenate([zeros4, hb], axis=0)
    w2t = jnp.concatenate([w_lo.T, w_hi.T], axis=0)

    ptt = perm.reshape(16, 16).astype(jnp.float32).T
    one = jnp.ones((1, 16), jnp.float32)

    k = math.gcd(bh, 4)
    grid = bh // k

    out = pl.pallas_call(
        _make_body(k, d, n),
        grid=(grid,),
        in_specs=[
            pl.BlockSpec((k * d, n), lambda i: (i, 0)),
            pl.BlockSpec((_NUM_PROJS, d), lambda i: (0, 0)),
            pl.BlockSpec((32, _NUM_PROJS), lambda i: (0, 0)),
            pl.BlockSpec((16, 16), lambda i: (0, 0)),
            pl.BlockSpec((1, 16), lambda i: (0, 0)),
        ],
        out_specs=pl.BlockSpec((k * n // 128, 128), lambda i: (i, 0)),
        out_shape=jax.ShapeDtypeStruct((m // 128, 128), jnp.int32),
        compiler_params=pltpu.CompilerParams(
            dimension_semantics=("arbitrary",),
        ),
    )(xt, pdt, w2t, ptt, one)
    return out.reshape(b, h, n)


# speedup vs baseline: 55.8363x; 1.0465x over previous
import math

import jax
import jax.numpy as jnp
from jax.experimental import pallas as pl
from jax.experimental.pallas import tpu as pltpu

_NUM_PROJS = 8


def _make_body(k, d, n):
    def _lsh_body(xt_ref, pdt_ref, w2t_ref, ptt_ref, one_ref, out_ref):
        pdt = pdt_ref[...]
        w2t = w2t_ref[...]
        ptt = ptt_ref[...]
        one = one_ref[...]
        for s in range(k):
            xt = xt_ref[pl.ds(s * d, d), :]
            yt = jnp.dot(pdt, xt, preferred_element_type=jnp.float32)
            pm = jnp.where(yt > 0.0, 1.0, -1.0)
            a = jnp.dot(w2t, pm, preferred_element_type=jnp.float32)
            oh = jnp.where(a == 4.0, 1.0, 0.0)
            rows = jnp.dot(ptt, oh[16:32, :],
                           preferred_element_type=jnp.float32)
            vals = jnp.dot(one, oh[0:16, :] * rows,
                           preferred_element_type=jnp.float32)
            ids = vals.astype(jnp.int32).reshape(n)
            out_ref[pl.ds(s * (n // 128), n // 128), :] = ids.reshape(
                n // 128, 128)
    return _lsh_body


def kernel(mat, proj_dir, perm):
    b, h, n, d = mat.shape
    m = b * h * n
    bh = b * h
    xt = mat.transpose(0, 1, 3, 2).reshape(bh * d, n)
    pdt = proj_dir.reshape(d, _NUM_PROJS).astype(jnp.float32).T

    nib = jnp.arange(16, dtype=jnp.int32)
    hb = (2 * ((nib[None, :] >> jnp.arange(4, dtype=jnp.int32)[:, None]) & 1)
          - 1).astype(jnp.float32)
    zeros4 = jnp.zeros((4, 16), jnp.float32)
    w_lo = jnp.concatenate([hb, zeros4], axis=0)
    w_hi = jnp.concatenate([zeros4, hb], axis=0)
    w2t = jnp.concatenate([w_lo.T, w_hi.T], axis=0)

    ptt = perm.reshape(16, 16).astype(jnp.float32).T
    one = jnp.ones((1, 16), jnp.float32)

    k = math.gcd(bh, 8)
    grid = bh // k

    out = pl.pallas_call(
        _make_body(k, d, n),
        grid=(grid,),
        in_specs=[
            pl.BlockSpec((k * d, n), lambda i: (i, 0)),
            pl.BlockSpec((_NUM_PROJS, d), lambda i: (0, 0)),
            pl.BlockSpec((32, _NUM_PROJS), lambda i: (0, 0)),
            pl.BlockSpec((16, 16), lambda i: (0, 0)),
            pl.BlockSpec((1, 16), lambda i: (0, 0)),
        ],
        out_specs=pl.BlockSpec((k * n // 128, 128), lambda i: (i, 0)),
        out_shape=jax.ShapeDtypeStruct((m // 128, 128), jnp.int32),
        compiler_params=pltpu.CompilerParams(
            dimension_semantics=("arbitrary",),
        ),
    )(xt, pdt, w2t, ptt, one)
    return out.reshape(b, h, n)
